# baseline (speedup 1.0000x reference)
import functools

import jax
import jax.numpy as jnp
from jax import lax
from jax.experimental import pallas as pl
from jax.experimental.pallas import tpu as pltpu
from jax.experimental.pallas import tpu_sc as plsc

_D = 64
_R = 128
_K = 8


@functools.lru_cache(maxsize=None)
def _make_gather(n_rows_total, g_per_w):
    info = plsc.get_sparse_core_info()
    nc, ns = info.num_cores, info.num_subcores
    nw = nc * ns
    mesh = plsc.VectorSubcoreMesh(core_axis_name="c", subcore_axis_name="s")

    @functools.partial(
        pl.kernel,
        out_type=jax.ShapeDtypeStruct((n_rows_total, _D), jnp.float32),
        mesh=mesh,
        compiler_params=pltpu.CompilerParams(use_tc_tiling_on_sc=False),
        scratch_types=[
            pltpu.VMEM((g_per_w, _R), jnp.int32),
            pltpu.VMEM((_K, _R, _D), jnp.float32),
            pltpu.SemaphoreType.DMA,
            pltpu.SemaphoreType.DMA,
        ],
    )
    def k(x_hbm, table_hbm, out_hbm, idx_v, rows_v, gsem, wsem):
        wid = lax.axis_index("s") * nc + lax.axis_index("c")
        pltpu.sync_copy(x_hbm.at[wid], idx_v)
        base = wid * (g_per_w * _R)

        def group(g, carry):
            gets = [
                pltpu.async_copy(
                    table_hbm.at[idx_v.at[g * _K + b]], rows_v.at[b], gsem)
                for b in range(_K)
            ]
            for h in gets:
                h.wait()
            puts = [
                pltpu.async_copy(
                    rows_v.at[b],
                    out_hbm.at[pl.ds(base + (g * _K + b) * _R, _R)], wsem)
                for b in range(_K)
            ]
            for h in puts:
                h.wait()
            return carry

        lax.fori_loop(0, g_per_w // _K, group, 0)

    return k, nw


def kernel(x, table):
    b, l = x.shape
    n = b * l
    info = plsc.get_sparse_core_info()
    nw = info.num_cores * info.num_subcores
    g_per_w = n // (nw * _R)
    fn, _ = _make_gather(n, g_per_w)
    x3d = x.astype(jnp.int32).reshape(nw, g_per_w, _R)
    out = fn(x3d, table)
    return out.reshape(b, l, _D)


# speedup vs baseline: 1.0116x; 1.0116x over previous
import functools

import jax
import jax.numpy as jnp
from jax import lax
from jax.experimental import pallas as pl
from jax.experimental.pallas import tpu as pltpu
from jax.experimental.pallas import tpu_sc as plsc

_D = 64
_R = 128
_NBUF = 8


@functools.lru_cache(maxsize=None)
def _make_gather(n_rows_total, g_per_w):
    info = plsc.get_sparse_core_info()
    nc, ns = info.num_cores, info.num_subcores
    mesh = plsc.VectorSubcoreMesh(core_axis_name="c", subcore_axis_name="s")
    nrounds = g_per_w // _NBUF
    assert g_per_w % _NBUF == 0

    @functools.partial(
        pl.kernel,
        out_type=jax.ShapeDtypeStruct((n_rows_total, _D), jnp.float32),
        mesh=mesh,
        compiler_params=pltpu.CompilerParams(use_tc_tiling_on_sc=False),
        scratch_types=[
            pltpu.VMEM((g_per_w, _R), jnp.int32),
            pltpu.VMEM((_NBUF, _R, _D), jnp.float32),
            pltpu.SemaphoreType.DMA((_NBUF,)),
            pltpu.SemaphoreType.DMA((_NBUF,)),
        ],
    )
    def k(x_hbm, table_hbm, out_hbm, idx_v, rows_v, gsems, wsems):
        wid = lax.axis_index("s") * nc + lax.axis_index("c")
        pltpu.sync_copy(x_hbm.at[wid], idx_v)
        base = wid * (g_per_w * _R)

        def gather(j, b):
            pltpu.async_copy(
                table_hbm.at[idx_v.at[j]], rows_v.at[b], gsems.at[b])

        def wait_gather(b):
            pltpu.make_async_copy(
                table_hbm.at[idx_v.at[0]], rows_v.at[b], gsems.at[b]).wait()

        def write(j, b):
            pltpu.async_copy(
                rows_v.at[b], out_hbm.at[pl.ds(base + j * _R, _R)],
                wsems.at[b])

        def wait_write(b):
            pltpu.make_async_copy(
                rows_v.at[b], out_hbm.at[pl.ds(base, _R)], wsems.at[b]).wait()

        for b in range(_NBUF):
            gather(b, b)

        def round_body(t, carry):
            j0 = t * _NBUF
            for b in range(_NBUF):
                wait_gather(b)
                write(j0 + b, b)
                wait_write(b)
                gather(j0 + b + _NBUF, b)
            return carry

        lax.fori_loop(0, nrounds - 1, round_body, 0)

        j0 = (nrounds - 1) * _NBUF
        for b in range(_NBUF):
            wait_gather(b)
            write(j0 + b, b)
        for b in range(_NBUF):
            wait_write(b)

    return k


def kernel(x, table):
    b, l = x.shape
    n = b * l
    info = plsc.get_sparse_core_info()
    nw = info.num_cores * info.num_subcores
    g_per_w = n // (nw * _R)
    fn = _make_gather(n, g_per_w)
    x3d = x.astype(jnp.int32).reshape(nw, g_per_w, _R)
    out = fn(x3d, table)
    return out.reshape(b, l, _D)
